# baseline (speedup 1.0000x reference)
import functools

import jax
import jax.numpy as jnp
from jax import lax
from jax.experimental import pallas as pl
from jax.experimental.pallas import tpu as pltpu
from jax.experimental.pallas import tpu_sc as plsc

NC = 2
NS = 16
L = 16
CW = 128
R = 256


def _sc_sum_body(x_hbm, out_hbm, buf0, buf1, acc_v, sem0, sem1, *, B, S, D):
    wid = lax.axis_index("s") * NC + lax.axis_index("c")
    col0 = wid * CW
    nch = S // R
    bufs = (buf0, buf1)
    sems = (sem0, sem1)
    nvec = CW // L

    def _src(b, ch):
        return x_hbm.at[b, pl.ds(ch * R, R), pl.ds(col0, CW)]

    def _rows(buf, a):
        def row_body(r, a):
            return tuple(a[c] + buf[r, pl.ds(c * L, L)] for c in range(nvec))
        return lax.fori_loop(0, R, row_body, a)

    for b in range(B):
        pltpu.async_copy(_src(b, 0), bufs[0], sems[0])
        pltpu.async_copy(_src(b, 1), bufs[1], sems[1])
        acc = tuple(jnp.zeros((L,), jnp.float32) for _ in range(nvec))

        def chunk_pair(i2, acc, b=b):
            for k in range(2):
                ch = i2 * 2 + k
                pltpu.make_async_copy(_src(b, ch), bufs[k], sems[k]).wait()
                acc = _rows(bufs[k], acc)
                pltpu.async_copy(_src(b, ch + 2), bufs[k], sems[k])
            return acc

        acc = lax.fori_loop(0, (nch - 2) // 2, chunk_pair, acc)
        for k in range(2):
            ch = nch - 2 + k
            pltpu.make_async_copy(_src(b, ch), bufs[k], sems[k]).wait()
            acc = _rows(bufs[k], acc)
        for c in range(nvec):
            acc_v[pl.ds(c * L, L)] = acc[c]
        pltpu.sync_copy(acc_v, out_hbm.at[b, pl.ds(col0, CW)])


def _finish_body(s_ref, w_ref, b_ref, out_ref, *, s_total):
    pooled = s_ref[...] * (1.0 / s_total)
    logits = jax.lax.dot_general(
        pooled, w_ref[...],
        dimension_numbers=(((1,), (1,)), ((), ())),
        preferred_element_type=jnp.float32,
    ) + b_ref[...]
    m = jnp.max(logits, axis=-1, keepdims=True)
    e = jnp.exp(logits - m)
    out_ref[...] = e / jnp.sum(e, axis=-1, keepdims=True)


def kernel(x, gate_weight, gate_bias):
    B, S, D = x.shape
    M = gate_weight.shape[0]

    mesh = plsc.VectorSubcoreMesh(core_axis_name="c", subcore_axis_name="s")
    sc_sum = functools.partial(
        pl.kernel,
        mesh=mesh,
        out_type=jax.ShapeDtypeStruct((B, D), jnp.float32),
        scratch_types=[
            pltpu.VMEM((R, CW), jnp.float32),
            pltpu.VMEM((R, CW), jnp.float32),
            pltpu.VMEM((CW,), jnp.float32),
            pltpu.SemaphoreType.DMA,
            pltpu.SemaphoreType.DMA,
        ],
    )(functools.partial(_sc_sum_body, B=B, S=S, D=D))
    sums = sc_sum(x)

    bias2d = gate_bias.reshape(1, M)
    return pl.pallas_call(
        functools.partial(_finish_body, s_total=S),
        in_specs=[
            pl.BlockSpec((B, D), lambda: (0, 0)),
            pl.BlockSpec((M, D), lambda: (0, 0)),
            pl.BlockSpec((1, M), lambda: (0, 0)),
        ],
        out_specs=pl.BlockSpec((B, M), lambda: (0, 0)),
        out_shape=jax.ShapeDtypeStruct((B, M), jnp.float32),
    )(sums, gate_weight, bias2d)


# speedup vs baseline: 1.3568x; 1.3568x over previous
import functools

import jax
import jax.numpy as jnp
from jax import lax
from jax.experimental import pallas as pl
from jax.experimental.pallas import tpu as pltpu
from jax.experimental.pallas import tpu_sc as plsc

NC = 2
NS = 16
L = 16
CW = 128
R = 256
S_SC = 3072
TC_BLK = 1024


def _sc_sum_body(x_hbm, out_hbm, buf0, buf1, acc_v, sem0, sem1, *, B, nch):
    wid = lax.axis_index("s") * NC + lax.axis_index("c")
    col0 = wid * CW
    bufs = (buf0, buf1)
    sems = (sem0, sem1)
    nvec = CW // L

    def _src(b, ch):
        return x_hbm.at[b, pl.ds(ch * R, R), pl.ds(col0, CW)]

    def _rows(buf, a):
        def row_body(r, a):
            return tuple(a[c] + buf[r, pl.ds(c * L, L)] for c in range(nvec))
        return lax.fori_loop(0, R, row_body, a)

    for b in range(B):
        pltpu.async_copy(_src(b, 0), bufs[0], sems[0])
        pltpu.async_copy(_src(b, 1), bufs[1], sems[1])
        acc = tuple(jnp.zeros((L,), jnp.float32) for _ in range(nvec))

        def chunk_pair(i2, acc, b=b):
            for k in range(2):
                ch = i2 * 2 + k
                pltpu.make_async_copy(_src(b, ch), bufs[k], sems[k]).wait()
                acc = _rows(bufs[k], acc)
                pltpu.async_copy(_src(b, ch + 2), bufs[k], sems[k])
            return acc

        acc = lax.fori_loop(0, (nch - 2) // 2, chunk_pair, acc)
        for k in range(2):
            ch = nch - 2 + k
            pltpu.make_async_copy(_src(b, ch), bufs[k], sems[k]).wait()
            acc = _rows(bufs[k], acc)
        for c in range(nvec):
            acc_v[pl.ds(c * L, L)] = acc[c]
        pltpu.sync_copy(acc_v, out_hbm.at[b, pl.ds(col0, CW)])


def _tc_sum_body(x_ref, out_ref, acc_ref, *, nsteps):
    b = pl.program_id(0)
    j = pl.program_id(1)

    @pl.when(j == 0)
    def _init():
        acc_ref[...] = jnp.zeros_like(acc_ref)

    acc_ref[...] += jnp.sum(x_ref[...], axis=1)

    @pl.when(j == nsteps - 1)
    def _emit():
        out_ref[pl.ds(b, 1), :] = acc_ref[...]


def _finish_body(a_ref, t_ref, w_ref, b_ref, out_ref, *, s_total):
    pooled = (a_ref[...] + t_ref[...]) * (1.0 / s_total)
    logits = jax.lax.dot_general(
        pooled, w_ref[...],
        dimension_numbers=(((1,), (1,)), ((), ())),
        preferred_element_type=jnp.float32,
    ) + b_ref[...]
    m = jnp.max(logits, axis=-1, keepdims=True)
    e = jnp.exp(logits - m)
    out_ref[...] = e / jnp.sum(e, axis=-1, keepdims=True)


def kernel(x, gate_weight, gate_bias):
    B, S, D = x.shape
    M = gate_weight.shape[0]

    mesh = plsc.VectorSubcoreMesh(core_axis_name="c", subcore_axis_name="s")
    sc_sum = functools.partial(
        pl.kernel,
        mesh=mesh,
        out_type=jax.ShapeDtypeStruct((B, D), jnp.float32),
        scratch_types=[
            pltpu.VMEM((R, CW), jnp.float32),
            pltpu.VMEM((R, CW), jnp.float32),
            pltpu.VMEM((CW,), jnp.float32),
            pltpu.SemaphoreType.DMA,
            pltpu.SemaphoreType.DMA,
        ],
    )(functools.partial(_sc_sum_body, B=B, nch=S_SC // R))
    sc_sums = sc_sum(x)

    j0 = S_SC // TC_BLK
    tc_steps = (S - S_SC) // TC_BLK
    tc_sums = pl.pallas_call(
        functools.partial(_tc_sum_body, nsteps=tc_steps),
        grid=(B, tc_steps),
        in_specs=[
            pl.BlockSpec((1, TC_BLK, D), lambda b, j: (b, j + j0, 0)),
        ],
        out_specs=pl.BlockSpec((B, D), lambda b, j: (0, 0)),
        out_shape=jax.ShapeDtypeStruct((B, D), jnp.float32),
        scratch_shapes=[pltpu.VMEM((1, D), jnp.float32)],
    )(x)

    bias2d = gate_bias.reshape(1, M)
    return pl.pallas_call(
        functools.partial(_finish_body, s_total=S),
        in_specs=[
            pl.BlockSpec((B, D), lambda: (0, 0)),
            pl.BlockSpec((B, D), lambda: (0, 0)),
            pl.BlockSpec((M, D), lambda: (0, 0)),
            pl.BlockSpec((1, M), lambda: (0, 0)),
        ],
        out_specs=pl.BlockSpec((B, M), lambda: (0, 0)),
        out_shape=jax.ShapeDtypeStruct((B, M), jnp.float32),
    )(sc_sums, tc_sums, gate_weight, bias2d)


# speedup vs baseline: 1.3886x; 1.0234x over previous
import functools

import jax
import jax.numpy as jnp
from jax import lax
from jax.experimental import pallas as pl
from jax.experimental.pallas import tpu as pltpu
from jax.experimental.pallas import tpu_sc as plsc

NC = 2
NS = 16
L = 16
CW = 128
R = 256
S_SC = 1024
TC_BLK = 1024


def _sc_sum_body(x_hbm, out_hbm, buf0, buf1, acc_v, sem0, sem1, *, B, nch):
    wid = lax.axis_index("s") * NC + lax.axis_index("c")
    col0 = wid * CW
    bufs = (buf0, buf1)
    sems = (sem0, sem1)
    nvec = CW // L

    def _src(b, ch):
        return x_hbm.at[b, pl.ds(ch * R, R), pl.ds(col0, CW)]

    def _rows(buf, a):
        def row_body(r, a):
            return tuple(a[c] + buf[r, pl.ds(c * L, L)] for c in range(nvec))
        return lax.fori_loop(0, R, row_body, a)

    for b in range(B):
        pltpu.async_copy(_src(b, 0), bufs[0], sems[0])
        pltpu.async_copy(_src(b, 1), bufs[1], sems[1])
        acc = tuple(jnp.zeros((L,), jnp.float32) for _ in range(nvec))

        def chunk_pair(i2, acc, b=b):
            for k in range(2):
                ch = i2 * 2 + k
                pltpu.make_async_copy(_src(b, ch), bufs[k], sems[k]).wait()
                acc = _rows(bufs[k], acc)
                pltpu.async_copy(_src(b, ch + 2), bufs[k], sems[k])
            return acc

        acc = lax.fori_loop(0, (nch - 2) // 2, chunk_pair, acc)
        for k in range(2):
            ch = nch - 2 + k
            pltpu.make_async_copy(_src(b, ch), bufs[k], sems[k]).wait()
            acc = _rows(bufs[k], acc)
        for c in range(nvec):
            acc_v[pl.ds(c * L, L)] = acc[c]
        pltpu.sync_copy(acc_v, out_hbm.at[b, pl.ds(col0, CW)])


def _tc_sum_body(x_ref, out_ref, acc_ref, *, nsteps):
    b = pl.program_id(0)
    j = pl.program_id(1)

    @pl.when(j == 0)
    def _init():
        acc_ref[...] = jnp.zeros_like(acc_ref)

    acc_ref[...] += jnp.sum(x_ref[...], axis=1)

    @pl.when(j == nsteps - 1)
    def _emit():
        out_ref[pl.ds(b, 1), :] = acc_ref[...]


def _finish_body(a_ref, t_ref, w_ref, b_ref, out_ref, *, s_total):
    pooled = (a_ref[...] + t_ref[...]) * (1.0 / s_total)
    logits = jax.lax.dot_general(
        pooled, w_ref[...],
        dimension_numbers=(((1,), (1,)), ((), ())),
        preferred_element_type=jnp.float32,
    ) + b_ref[...]
    m = jnp.max(logits, axis=-1, keepdims=True)
    e = jnp.exp(logits - m)
    out_ref[...] = e / jnp.sum(e, axis=-1, keepdims=True)


def kernel(x, gate_weight, gate_bias):
    B, S, D = x.shape
    M = gate_weight.shape[0]

    mesh = plsc.VectorSubcoreMesh(core_axis_name="c", subcore_axis_name="s")
    sc_sum = functools.partial(
        pl.kernel,
        mesh=mesh,
        out_type=jax.ShapeDtypeStruct((B, D), jnp.float32),
        scratch_types=[
            pltpu.VMEM((R, CW), jnp.float32),
            pltpu.VMEM((R, CW), jnp.float32),
            pltpu.VMEM((CW,), jnp.float32),
            pltpu.SemaphoreType.DMA,
            pltpu.SemaphoreType.DMA,
        ],
    )(functools.partial(_sc_sum_body, B=B, nch=S_SC // R))
    sc_sums = sc_sum(x)

    j0 = S_SC // TC_BLK
    tc_steps = (S - S_SC) // TC_BLK
    tc_sums = pl.pallas_call(
        functools.partial(_tc_sum_body, nsteps=tc_steps),
        grid=(B, tc_steps),
        in_specs=[
            pl.BlockSpec((1, TC_BLK, D), lambda b, j: (b, j + j0, 0)),
        ],
        out_specs=pl.BlockSpec((B, D), lambda b, j: (0, 0)),
        out_shape=jax.ShapeDtypeStruct((B, D), jnp.float32),
        scratch_shapes=[pltpu.VMEM((1, D), jnp.float32)],
    )(x)

    bias2d = gate_bias.reshape(1, M)
    return pl.pallas_call(
        functools.partial(_finish_body, s_total=S),
        in_specs=[
            pl.BlockSpec((B, D), lambda: (0, 0)),
            pl.BlockSpec((B, D), lambda: (0, 0)),
            pl.BlockSpec((M, D), lambda: (0, 0)),
            pl.BlockSpec((1, M), lambda: (0, 0)),
        ],
        out_specs=pl.BlockSpec((B, M), lambda: (0, 0)),
        out_shape=jax.ShapeDtypeStruct((B, M), jnp.float32),
    )(sc_sums, tc_sums, gate_weight, bias2d)


# speedup vs baseline: 1.5540x; 1.1191x over previous
import functools

import jax
import jax.numpy as jnp
from jax.experimental import pallas as pl
from jax.experimental.pallas import tpu as pltpu


def _body(x_ref, w_ref, b_ref, out_ref, acc_ref, *, nsteps, s_total):
    b = pl.program_id(0)
    j = pl.program_id(1)

    @pl.when(j == 0)
    def _init():
        acc_ref[...] = jnp.zeros_like(acc_ref)

    acc_ref[...] += jnp.sum(x_ref[...], axis=1)

    @pl.when(j == nsteps - 1)
    def _finish():
        pooled = acc_ref[...] * (1.0 / s_total)
        logits = jax.lax.dot_general(
            pooled, w_ref[...],
            dimension_numbers=(((1,), (1,)), ((), ())),
            preferred_element_type=jnp.float32,
        ) + b_ref[...]
        m = jnp.max(logits, axis=-1, keepdims=True)
        e = jnp.exp(logits - m)
        out_ref[pl.ds(b, 1), :] = e / jnp.sum(e, axis=-1, keepdims=True)


def kernel(x, gate_weight, gate_bias):
    B, S, D = x.shape
    M = gate_weight.shape[0]
    s_blk = 512
    while S % s_blk != 0:
        s_blk //= 2
    nsteps = S // s_blk

    bias2d = gate_bias.reshape(1, M)

    return pl.pallas_call(
        functools.partial(_body, nsteps=nsteps, s_total=S),
        grid=(B, nsteps),
        in_specs=[
            pl.BlockSpec((1, s_blk, D), lambda b, j: (b, j, 0)),
            pl.BlockSpec((M, D), lambda b, j: (0, 0)),
            pl.BlockSpec((1, M), lambda b, j: (0, 0)),
        ],
        out_specs=pl.BlockSpec((B, M), lambda b, j: (0, 0)),
        out_shape=jax.ShapeDtypeStruct((B, M), jnp.float32),
        scratch_shapes=[pltpu.VMEM((1, D), jnp.float32)],
    )(x, gate_weight, bias2d)


# speedup vs baseline: 1.5565x; 1.0016x over previous
import functools

import jax
import jax.numpy as jnp
from jax.experimental import pallas as pl
from jax.experimental.pallas import tpu as pltpu


def _body(x_ref, w_ref, b_ref, out_ref, lacc_ref, *, nsteps, s_total):
    b = pl.program_id(0)
    j = pl.program_id(1)

    @pl.when(jnp.logical_and(b == 0, j == 0))
    def _init():
        lacc_ref[...] = jnp.zeros_like(lacc_ref)

    part = jnp.sum(x_ref[...], axis=1)
    lacc_ref[pl.ds(b, 1), :] += jax.lax.dot_general(
        part, w_ref[...],
        dimension_numbers=(((1,), (1,)), ((), ())),
        preferred_element_type=jnp.float32,
    )

    @pl.when(jnp.logical_and(b == pl.num_programs(0) - 1, j == nsteps - 1))
    def _finish():
        logits = lacc_ref[...] * (1.0 / s_total) + b_ref[...]
        m = jnp.max(logits, axis=-1, keepdims=True)
        e = jnp.exp(logits - m)
        out_ref[...] = e / jnp.sum(e, axis=-1, keepdims=True)


def kernel(x, gate_weight, gate_bias):
    B, S, D = x.shape
    M = gate_weight.shape[0]
    s_blk = 1024
    while S % s_blk != 0:
        s_blk //= 2
    nsteps = S // s_blk

    bias2d = gate_bias.reshape(1, M)

    return pl.pallas_call(
        functools.partial(_body, nsteps=nsteps, s_total=S),
        grid=(B, nsteps),
        in_specs=[
            pl.BlockSpec((1, s_blk, D), lambda b, j: (b, j, 0)),
            pl.BlockSpec((M, D), lambda b, j: (0, 0)),
            pl.BlockSpec((1, M), lambda b, j: (0, 0)),
        ],
        out_specs=pl.BlockSpec((B, M), lambda b, j: (0, 0)),
        out_shape=jax.ShapeDtypeStruct((B, M), jnp.float32),
        scratch_shapes=[pltpu.VMEM((B, M), jnp.float32)],
    )(x, gate_weight, bias2d)


# speedup vs baseline: 1.5574x; 1.0006x over previous
import functools

import jax
import jax.numpy as jnp
from jax.experimental import pallas as pl
from jax.experimental.pallas import tpu as pltpu


def _body(x_ref, w_ref, b_ref, out_ref, acc_ref, *, nsteps, s_total):
    b = pl.program_id(0)
    j = pl.program_id(1)

    @pl.when(j == 0)
    def _init():
        acc_ref[...] = jnp.zeros_like(acc_ref)

    acc_ref[...] += jnp.sum(x_ref[...], axis=1)

    @pl.when(j == nsteps - 1)
    def _finish():
        pooled = acc_ref[...] * (1.0 / s_total)
        logits = jax.lax.dot_general(
            pooled, w_ref[...],
            dimension_numbers=(((1,), (1,)), ((), ())),
            preferred_element_type=jnp.float32,
        ) + b_ref[...]
        m = jnp.max(logits, axis=-1, keepdims=True)
        e = jnp.exp(logits - m)
        out_ref[pl.ds(b, 1), :] = e / jnp.sum(e, axis=-1, keepdims=True)


def kernel(x, gate_weight, gate_bias):
    B, S, D = x.shape
    M = gate_weight.shape[0]
    s_blk = 1024
    while S % s_blk != 0:
        s_blk //= 2
    nsteps = S // s_blk

    bias2d = gate_bias.reshape(1, M)

    return pl.pallas_call(
        functools.partial(_body, nsteps=nsteps, s_total=S),
        grid=(B, nsteps),
        in_specs=[
            pl.BlockSpec((1, s_blk, D), lambda b, j: (b, j, 0)),
            pl.BlockSpec((M, D), lambda b, j: (0, 0)),
            pl.BlockSpec((1, M), lambda b, j: (0, 0)),
        ],
        out_specs=pl.BlockSpec((B, M), lambda b, j: (0, 0)),
        out_shape=jax.ShapeDtypeStruct((B, M), jnp.float32),
        scratch_shapes=[pltpu.VMEM((1, D), jnp.float32)],
    )(x, gate_weight, bias2d)
